# baseline (speedup 1.0000x reference)
import jax
import jax.numpy as jnp
from jax.experimental import pallas as pl

B = 8
N = 1250
S1, R1, K1 = 250, 0.2, 64
S2, R2, K2 = 62, 0.4, 64
OUT = 3


def _apply_mlp(ps, x):
    for i in range(len(ps) // 2):
        x = x @ ps[2 * i] + ps[2 * i + 1]
        x = jax.nn.relu(x)
    return x


def _fps(pos, n_sample):
    p = pos
    n = p.shape[0]

    def body(carry, _):
        mind, last = carry
        d = jnp.sum((p - p[last]) ** 2, axis=-1)
        mind = jnp.minimum(mind, d)
        nxt = jnp.argmax(mind).astype(jnp.int32)
        return (mind, nxt), nxt

    (_, _), idx_rest = jax.lax.scan(
        body, (jnp.full((n,), jnp.inf, dtype=jnp.float32), jnp.int32(0)),
        None, length=n_sample - 1)
    return jnp.concatenate([jnp.zeros((1,), dtype=jnp.int32), idx_rest])


def _radius_group(pos_c, pos, r, K):
    d2 = jnp.sum((pos_c[:, None, :] - pos[None, :, :]) ** 2, axis=-1)
    score = jnp.where(d2 <= r * r, -d2, -jnp.inf)
    vals, idx = jax.lax.top_k(score, K)
    return idx, vals > -jnp.inf


def _sa_module(ps, x, pos, n_sample, r, K):
    idx_s = _fps(pos, n_sample)
    pos_c = pos[idx_s]
    nbr, valid = _radius_group(pos_c, pos, r, K)
    rel = pos[nbr] - pos_c[:, None, :]
    feat = rel if x is None else jnp.concatenate([x[nbr], rel], axis=-1)
    h = _apply_mlp(ps, feat)
    h = jnp.where(valid[:, :, None], h, -jnp.inf)
    return jnp.max(h, axis=1), pos_c


def _knn_interp(x_c, pos_c, pos_f, k):
    k = min(k, pos_c.shape[0])
    d2 = jnp.sum((pos_f[:, None, :] - pos_c[None, :, :]) ** 2, axis=-1)
    vals, idx = jax.lax.top_k(-d2, k)
    w = 1.0 / jnp.maximum(-vals, 1e-16)
    w = w / jnp.sum(w, axis=-1, keepdims=True)
    return jnp.sum(x_c[idx] * w[:, :, None], axis=1)


def _fp_module(ps, x_c, pos_c, x_skip, pos_f, k=3):
    xi = _knn_interp(x_c, pos_c, pos_f, k)
    if x_skip is not None:
        xi = jnp.concatenate([xi, x_skip], axis=-1)
    return _apply_mlp(ps, xi)


def _head_kernel(f1_ref, w1_ref, b1_ref, w2_ref, b2_ref, w3_ref, b3_ref,
                 out_ref):
    h = jnp.maximum(f1_ref[...] @ w1_ref[...] + b1_ref[...], 0.0)
    h = jnp.maximum(h @ w2_ref[...] + b2_ref[...], 0.0)
    out_ref[...] = h @ w3_ref[...] + b3_ref[...]


def _head(f1, params):
    M = f1.shape[0]
    w3 = jnp.zeros((128, 128), f1.dtype).at[:, :OUT].set(params['lin3'][0])
    b3 = jnp.zeros((128,), f1.dtype).at[:OUT].set(params['lin3'][1])
    out = pl.pallas_call(
        _head_kernel,
        out_shape=jax.ShapeDtypeStruct((M, 128), f1.dtype),
    )(f1, params['lin1'][0], params['lin1'][1][None, :],
      params['lin2'][0], params['lin2'][1][None, :], w3, b3[None, :])
    return out[:, :OUT]


def _forward_cloud(params, pos):
    x1, pos1 = _sa_module(params['sa1'], None, pos, S1, R1, K1)
    x2, pos2 = _sa_module(params['sa2'], x1, pos1, S2, R2, K2)
    xg = jnp.max(_apply_mlp(params['gsa'],
                            jnp.concatenate([x2, pos2], axis=-1)),
                 axis=0, keepdims=True)
    posg = jnp.zeros((1, 3), dtype=pos.dtype)
    f3 = _fp_module(params['fp3'], xg, posg, x2, pos2, k=1)
    f2 = _fp_module(params['fp2'], f3, pos2, x1, pos1, k=3)
    f1 = _fp_module(params['fp1'], f2, pos1, None, pos, k=3)
    return f1


def kernel(pos, batch, params):
    pb = pos.reshape(B, N, 3)
    f1 = jax.vmap(lambda p: _forward_cloud(params, p))(pb)
    return _head(f1.reshape(B * N, 128), params)


# speedup vs baseline: 8.7850x; 8.7850x over previous
import functools

import jax
import jax.numpy as jnp
from jax.experimental import pallas as pl

B = 8
N = 1250
NP = 1280
S1, R1, K1 = 250, 0.2, 64
SP1 = 256
S2, R2, K2 = 62, 0.4, 64
SP2 = 64
OUT = 3
NEG = -1e30
BIGD = 1e30
FAR = 1e6


def _iota(shape, dim):
    return jax.lax.broadcasted_iota(jnp.int32, shape, dim)


def _argext(score, ext, col, bigc):
    cand = jnp.where(score == ext, col, bigc)
    return jnp.min(cand, axis=1, keepdims=True)


def _fps_level(px, py, pz, n_src, n_sample, out_lanes):
    C = px.shape[1]
    colmask = _iota((B, C), 1) < n_src
    lane_out = _iota((B, out_lanes), 1)
    col = _iota((B, C), 1)

    lx = px[:, 0:1]
    ly = py[:, 0:1]
    lz = pz[:, 0:1]
    xc = jnp.where(lane_out == 0, lx, FAR)
    yc = jnp.where(lane_out == 0, ly, FAR)
    zc = jnp.where(lane_out == 0, lz, FAR)
    mind = jnp.full((B, C), BIGD, dtype=jnp.float32)

    def body(t, carry):
        mind, lx, ly, lz, xc, yc, zc = carry
        d = (px - lx) ** 2 + (py - ly) ** 2 + (pz - lz) ** 2
        mind = jnp.where(colmask, jnp.minimum(mind, d), NEG)
        m = jnp.max(mind, axis=1, keepdims=True)
        idx = _argext(mind, m, col, C)
        sel = col == idx
        lx = jnp.sum(jnp.where(sel, px, 0.0), axis=1, keepdims=True)
        ly = jnp.sum(jnp.where(sel, py, 0.0), axis=1, keepdims=True)
        lz = jnp.sum(jnp.where(sel, pz, 0.0), axis=1, keepdims=True)
        put = lane_out == t
        xc = jnp.where(put, lx, xc)
        yc = jnp.where(put, ly, yc)
        zc = jnp.where(put, lz, zc)
        return mind, lx, ly, lz, xc, yc, zc

    carry = (mind, lx, ly, lz, xc, yc, zc)
    carry = jax.lax.fori_loop(1, n_sample, body, carry)
    return carry[4], carry[5], carry[6]


def _fps_kernel(px_ref, py_ref, pz_ref,
                xc1_ref, yc1_ref, zc1_ref, xc2_ref, yc2_ref, zc2_ref):
    px, py, pz = px_ref[...], py_ref[...], pz_ref[...]
    xc1, yc1, zc1 = _fps_level(px, py, pz, N, S1, SP1)
    xc1_ref[...], yc1_ref[...], zc1_ref[...] = xc1, yc1, zc1
    xc2, yc2, zc2 = _fps_level(xc1, yc1, zc1, S1, S2, SP2)
    xc2_ref[...], yc2_ref[...], zc2_ref[...] = xc2, yc2, zc2


def _sa_loop(score, G, Gc, b1, W2, b2, W3, b3, n_k, out_w):
    R, C = score.shape
    col = _iota((R, C), 1)
    acc = jnp.full((R, out_w), NEG, dtype=jnp.float32)
    for _ in range(n_k):
        m = jnp.max(score, axis=1, keepdims=True)
        validf = m > -1e29
        idx = _argext(score, m, col, C)
        onehot = col == idx
        score = jnp.where(onehot, NEG, score)
        gsel = jax.lax.dot(onehot.astype(jnp.float32), G,
                           preferred_element_type=jnp.float32)
        h = jnp.maximum(gsel - Gc + b1, 0.0)
        h = jnp.maximum(jax.lax.dot(h, W2,
                                    preferred_element_type=jnp.float32)
                        + b2, 0.0)
        h = jnp.maximum(jax.lax.dot(h, W3,
                                    preferred_element_type=jnp.float32)
                        + b3, 0.0)
        acc = jnp.maximum(acc, jnp.where(validf, h, NEG))
    return acc


def _sa1_kernel(px_ref, py_ref, pz_ref, prow_ref, crow_ref,
                w1_ref, b1_ref, w2_ref, b2_ref, w3_ref, b3_ref,
                x1_ref):
    px, py, pz = px_ref[0], py_ref[0], pz_ref[0]
    prow = prow_ref[...]
    crow = crow_ref[...]
    xc = crow[:, 0:1]
    yc = crow[:, 1:2]
    zc = crow[:, 2:3]
    d2 = (xc - px) ** 2 + (yc - py) ** 2 + (zc - pz) ** 2
    ok = (d2 <= R1 * R1) & (_iota((SP1, NP), 1) < N)
    score = jnp.where(ok, -d2, NEG)

    G = jax.lax.dot(prow, w1_ref[...],
                    preferred_element_type=jnp.float32)
    Gc = jax.lax.dot(crow, w1_ref[...],
                     preferred_element_type=jnp.float32)
    x1 = _sa_loop(score, G, Gc, b1_ref[...], w2_ref[...], b2_ref[...],
                  w3_ref[...], b3_ref[...], K1, 128)
    x1 = jnp.where(_iota((SP1, 128), 0) < S1, x1, 0.0)
    x1_ref[...] = x1


def _sa2_kernel(xs_ref, ys_ref, zs_ref, srow_ref, crow_ref, x1_ref,
                w1x_ref, w1p_ref, b1_ref, w2_ref, b2_ref, w3_ref, b3_ref,
                x2_ref):
    xs, ys, zs = xs_ref[0], ys_ref[0], zs_ref[0]
    srow = srow_ref[...]
    crow = crow_ref[...]
    xc = crow[:, 0:1]
    yc = crow[:, 1:2]
    zc = crow[:, 2:3]
    d2 = (xc - xs) ** 2 + (yc - ys) ** 2 + (zc - zs) ** 2
    ok = (d2 <= R2 * R2) & (_iota((SP2, SP1), 1) < S1)
    score = jnp.where(ok, -d2, NEG)

    z = (jax.lax.dot(x1_ref[...], w1x_ref[...],
                     preferred_element_type=jnp.float32)
         + jax.lax.dot(srow, w1p_ref[...],
                       preferred_element_type=jnp.float32))
    Gc = jax.lax.dot(crow, w1p_ref[...],
                     preferred_element_type=jnp.float32)
    x2 = _sa_loop(score, z, Gc, b1_ref[...], w2_ref[...], b2_ref[...],
                  w3_ref[...], b3_ref[...], K2, 256)
    x2 = jnp.where(_iota((SP2, 256), 0) < S2, x2, 0.0)
    x2_ref[...] = x2


def _knn3_weights(d2, n_src):
    R, C = d2.shape
    col = _iota((R, C), 1)
    d2 = jnp.where(col < n_src, d2, BIGD)
    wmat = jnp.zeros((R, C), dtype=jnp.float32)
    wsum = jnp.zeros((R, 1), dtype=jnp.float32)
    for _ in range(3):
        m = jnp.min(d2, axis=1, keepdims=True)
        idx = _argext(d2, m, col, C)
        onehot = col == idx
        w = 1.0 / jnp.maximum(m, 1e-16)
        wmat = jnp.where(onehot, w, wmat)
        wsum = wsum + w
        d2 = jnp.where(onehot, BIGD, d2)
    return wmat / wsum


def _tail_kernel(x2_ref, c2row_ref, x1_ref, c1row_ref,
                 xs1_ref, ys1_ref, zs1_ref, xs2_ref, ys2_ref, zs2_ref,
                 prow_ref,
                 gw1x_ref, gw1p_ref, gb1_ref, gw2_ref, gb2_ref,
                 gw3_ref, gb3_ref,
                 f3wg_ref, f3wx_ref, f3b1_ref, f3w2_ref, f3b2_ref,
                 f2wi_ref, f2ws_ref, f2b1_ref, f2w2_ref, f2b2_ref,
                 f1w1_ref, f1b1_ref, f1w2_ref, f1b2_ref, f1w3_ref,
                 f1b3_ref,
                 l1w_ref, l1b_ref, l2w_ref, l2b_ref, l3w_ref, l3b_ref,
                 out_ref):
    x2 = x2_ref[...]
    c2row = c2row_ref[...]
    x1 = x1_ref[...]
    c1row = c1row_ref[...]
    prow = prow_ref[...]

    h = jnp.maximum(jax.lax.dot(x2, gw1x_ref[...],
                                preferred_element_type=jnp.float32)
                    + jax.lax.dot(c2row, gw1p_ref[...],
                                  preferred_element_type=jnp.float32)
                    + gb1_ref[...], 0.0)
    h = jnp.maximum(jax.lax.dot(h, gw2_ref[...],
                                preferred_element_type=jnp.float32)
                    + gb2_ref[...], 0.0)
    h = jnp.maximum(jax.lax.dot(h, gw3_ref[...],
                                preferred_element_type=jnp.float32)
                    + gb3_ref[...], 0.0)
    h = jnp.where(_iota(h.shape, 0) < S2, h, NEG)
    xg = jnp.max(h, axis=0, keepdims=True)

    h = jnp.maximum(jax.lax.dot(xg, f3wg_ref[...],
                                preferred_element_type=jnp.float32)
                    + jax.lax.dot(x2, f3wx_ref[...],
                                  preferred_element_type=jnp.float32)
                    + f3b1_ref[...], 0.0)
    f3 = jnp.maximum(jax.lax.dot(h, f3w2_ref[...],
                                 preferred_element_type=jnp.float32)
                     + f3b2_ref[...], 0.0)

    xc = c1row[:, 0:1]
    yc = c1row[:, 1:2]
    zc = c1row[:, 2:3]
    d2 = ((xc - xs2_ref[0]) ** 2 + (yc - ys2_ref[0]) ** 2
          + (zc - zs2_ref[0]) ** 2)
    wmat = _knn3_weights(d2, S2)
    xi = jax.lax.dot(wmat, f3, preferred_element_type=jnp.float32)
    h = jnp.maximum(jax.lax.dot(xi, f2wi_ref[...],
                                preferred_element_type=jnp.float32)
                    + jax.lax.dot(x1, f2ws_ref[...],
                                  preferred_element_type=jnp.float32)
                    + f2b1_ref[...], 0.0)
    f2 = jnp.maximum(jax.lax.dot(h, f2w2_ref[...],
                                 preferred_element_type=jnp.float32)
                     + f2b2_ref[...], 0.0)

    xp = prow[:, 0:1]
    yp = prow[:, 1:2]
    zp = prow[:, 2:3]
    d2 = ((xp - xs1_ref[0]) ** 2 + (yp - ys1_ref[0]) ** 2
          + (zp - zs1_ref[0]) ** 2)
    wmat = _knn3_weights(d2, S1)
    h = jax.lax.dot(wmat, f2, preferred_element_type=jnp.float32)
    h = jnp.maximum(jax.lax.dot(h, f1w1_ref[...],
                                preferred_element_type=jnp.float32)
                    + f1b1_ref[...], 0.0)
    h = jnp.maximum(jax.lax.dot(h, f1w2_ref[...],
                                preferred_element_type=jnp.float32)
                    + f1b2_ref[...], 0.0)
    h = jnp.maximum(jax.lax.dot(h, f1w3_ref[...],
                                preferred_element_type=jnp.float32)
                    + f1b3_ref[...], 0.0)

    h = jnp.maximum(jax.lax.dot(h, l1w_ref[...],
                                preferred_element_type=jnp.float32)
                    + l1b_ref[...], 0.0)
    h = jnp.maximum(jax.lax.dot(h, l2w_ref[...],
                                preferred_element_type=jnp.float32)
                    + l2b_ref[...], 0.0)
    out_ref[...] = (jax.lax.dot(h, l3w_ref[...],
                                preferred_element_type=jnp.float32)
                    + l3b_ref[...])


def _full(shape, dtype=jnp.float32):
    return pl.BlockSpec(shape, lambda i: (0, 0))


def _row(shape):
    return pl.BlockSpec(shape, lambda i: (i, 0))


def _lane(c):
    return pl.BlockSpec((1, 1, c), lambda i: (i, 0, 0))


def _padw(w, rows):
    return jnp.zeros((rows, w.shape[1]), jnp.float32).at[:w.shape[0]].set(w)


def kernel(pos, batch, params):
    f32 = jnp.float32
    pb = pos.reshape(B, N, 3)
    pad = jnp.full((B, NP - N, 3), FAR, f32)
    pbp = jnp.concatenate([pb, pad], axis=1)
    px = pbp[:, :, 0]
    py = pbp[:, :, 1]
    pz = pbp[:, :, 2]
    prow = jnp.concatenate(
        [pbp, jnp.zeros((B, NP, 5), f32)], axis=2).reshape(B * NP, 8)

    fps_out = pl.pallas_call(
        _fps_kernel,
        out_shape=[jax.ShapeDtypeStruct((B, SP1), f32)] * 3
        + [jax.ShapeDtypeStruct((B, SP2), f32)] * 3,
    )(px, py, pz)
    xc1, yc1, zc1, xc2, yc2, zc2 = fps_out
    px3 = px.reshape(B, 1, NP)
    py3 = py.reshape(B, 1, NP)
    pz3 = pz.reshape(B, 1, NP)
    xc13 = xc1.reshape(B, 1, SP1)
    yc13 = yc1.reshape(B, 1, SP1)
    zc13 = zc1.reshape(B, 1, SP1)
    xc23 = xc2.reshape(B, 1, SP2)
    yc23 = yc2.reshape(B, 1, SP2)
    zc23 = zc2.reshape(B, 1, SP2)
    c1row = jnp.concatenate(
        [jnp.stack([xc1, yc1, zc1], axis=2),
         jnp.zeros((B, SP1, 5), f32)], axis=2).reshape(B * SP1, 8)
    c2row = jnp.concatenate(
        [jnp.stack([xc2, yc2, zc2], axis=2),
         jnp.zeros((B, SP2, 5), f32)], axis=2).reshape(B * SP2, 8)

    p = params
    sa1w1 = _padw(p['sa1'][0], 8)
    sa1 = pl.pallas_call(
        _sa1_kernel,
        grid=(B,),
        in_specs=[
            _lane(NP), _lane(NP), _lane(NP),
            _row((NP, 8)), _row((SP1, 8)),
            _full((8, 64)), _full((1, 64)), _full((64, 64)),
            _full((1, 64)), _full((64, 128)), _full((1, 128)),
        ],
        out_specs=_row((SP1, 128)),
        out_shape=jax.ShapeDtypeStruct((B * SP1, 128), f32),
    )(px3, py3, pz3, prow, c1row,
      sa1w1, p['sa1'][1][None], p['sa1'][2], p['sa1'][3][None],
      p['sa1'][4], p['sa1'][5][None])

    sa2w1x = p['sa2'][0][:128]
    sa2w1p = _padw(p['sa2'][0][128:], 8)
    sa2 = pl.pallas_call(
        _sa2_kernel,
        grid=(B,),
        in_specs=[
            _lane(SP1), _lane(SP1), _lane(SP1),
            _row((SP1, 8)), _row((SP2, 8)), _row((SP1, 128)),
            _full((128, 128)), _full((8, 128)), _full((1, 128)),
            _full((128, 128)), _full((1, 128)),
            _full((128, 256)), _full((1, 256)),
        ],
        out_specs=_row((SP2, 256)),
        out_shape=jax.ShapeDtypeStruct((B * SP2, 256), f32),
    )(xc13, yc13, zc13, c1row, c2row, sa1,
      sa2w1x, sa2w1p, p['sa2'][1][None], p['sa2'][2], p['sa2'][3][None],
      p['sa2'][4], p['sa2'][5][None])

    gsa1x = p['gsa'][0][:256]
    gsa1p = _padw(p['gsa'][0][256:], 8)
    f3wg = p['fp3'][0][:1024]
    f3wx = p['fp3'][0][1024:]
    f2wi = p['fp2'][0][:256]
    f2ws = p['fp2'][0][256:]
    l3w = jnp.zeros((128, 128), f32).at[:, :OUT].set(p['lin3'][0])
    l3b = jnp.zeros((1, 128), f32).at[0, :OUT].set(p['lin3'][1])

    weights = [
        (gsa1x, (256, 256)), (gsa1p, (8, 256)),
        (p['gsa'][1][None], (1, 256)), (p['gsa'][2], (256, 512)),
        (p['gsa'][3][None], (1, 512)), (p['gsa'][4], (512, 1024)),
        (p['gsa'][5][None], (1, 1024)),
        (f3wg, (1024, 256)), (f3wx, (256, 256)),
        (p['fp3'][1][None], (1, 256)), (p['fp3'][2], (256, 256)),
        (p['fp3'][3][None], (1, 256)),
        (f2wi, (256, 256)), (f2ws, (128, 256)),
        (p['fp2'][1][None], (1, 256)), (p['fp2'][2], (256, 128)),
        (p['fp2'][3][None], (1, 128)),
        (p['fp1'][0], (128, 128)), (p['fp1'][1][None], (1, 128)),
        (p['fp1'][2], (128, 128)), (p['fp1'][3][None], (1, 128)),
        (p['fp1'][4], (128, 128)), (p['fp1'][5][None], (1, 128)),
        (p['lin1'][0], (128, 128)), (p['lin1'][1][None], (1, 128)),
        (p['lin2'][0], (128, 128)), (p['lin2'][1][None], (1, 128)),
        (l3w, (128, 128)), (l3b, (1, 128)),
    ]
    out = pl.pallas_call(
        _tail_kernel,
        grid=(B,),
        in_specs=[
            _row((SP2, 256)), _row((SP2, 8)), _row((SP1, 128)),
            _row((SP1, 8)),
            _lane(SP1), _lane(SP1), _lane(SP1),
            _lane(SP2), _lane(SP2), _lane(SP2),
            _row((NP, 8)),
        ] + [_full(s) for _, s in weights],
        out_specs=_row((NP, 128)),
        out_shape=jax.ShapeDtypeStruct((B * NP, 128), f32),
    )(sa2, c2row, sa1, c1row, xc13, yc13, zc13, xc23, yc23, zc23, prow,
      *[w for w, _ in weights])

    return out.reshape(B, NP, 128)[:, :N, :OUT].reshape(B * N, OUT)
